# baseline (speedup 1.0000x reference)
import functools

import jax
import jax.numpy as jnp
from jax.experimental import pallas as pl
from jax.experimental.pallas import tpu as pltpu


TILE_N = 1024


def _body(xyz_ref, feat_ref, w1x_ref, w1f_ref, b1_ref, w2_ref, b2_ref,
          w3_ref, b3_ref, out_ref, acc_ref):
    n = pl.program_id(1)
    num_n = pl.num_programs(1)
    x = feat_ref[0].astype(jnp.bfloat16)
    xyzt = xyz_ref[0].astype(jnp.bfloat16)

    dims = (((1,), (1,)), ((), ()))
    h1 = jax.lax.dot_general(x, w1f_ref[...], dims,
                             preferred_element_type=jnp.float32)
    h1 += jax.lax.dot_general(xyzt, w1x_ref[...], dims,
                              preferred_element_type=jnp.float32)
    h1 = jnp.maximum(h1.astype(jnp.bfloat16) + b1_ref[...], 0.0)

    h2 = jax.lax.dot_general(h1, w2_ref[...], dims,
                             preferred_element_type=jnp.float32)
    h2 = jnp.maximum(h2.astype(jnp.bfloat16) + b2_ref[...], 0.0)

    h3 = jax.lax.dot_general(h2, w3_ref[...], dims,
                             preferred_element_type=jnp.float32)

    tile_max = jnp.max(h3, axis=0, keepdims=True).astype(jnp.bfloat16)

    @pl.when(n == 0)
    def _init():
        acc_ref[...] = tile_max

    @pl.when(n != 0)
    def _acc():
        acc_ref[...] = jnp.maximum(acc_ref[...], tile_max)

    @pl.when(n == num_n - 1)
    def _finish():
        m = acc_ref[...].astype(jnp.float32)
        out_ref[0] = jnp.maximum(m + b3_ref[...], 0.0)


@functools.partial(jax.jit, static_argnames=())
def kernel(xyz, features, W1, b1, W2, b2, W3, b3):
    B, N, C = features.shape
    Cout = W3.shape[0]
    T = TILE_N

    w1x = W1[:, :3].astype(jnp.bfloat16)
    w1f = W1[:, 3:].astype(jnp.bfloat16)
    w2 = W2.astype(jnp.bfloat16)
    w3 = W3.astype(jnp.bfloat16)
    b1r = b1.reshape(1, -1).astype(jnp.bfloat16)
    b2r = b2.reshape(1, -1).astype(jnp.bfloat16)
    b3r = b3.reshape(1, -1)

    rep = lambda shape: pl.BlockSpec(shape, lambda b, n: (0,) * len(shape))

    out = pl.pallas_call(
        _body,
        grid=(B, N // T),
        in_specs=[
            pl.BlockSpec((1, T, 3), lambda b, n: (b, n, 0)),
            pl.BlockSpec((1, T, C), lambda b, n: (b, n, 0)),
            rep(w1x.shape),
            rep(w1f.shape),
            rep((1, w1f.shape[0])),
            rep(w2.shape),
            rep((1, w2.shape[0])),
            rep(w3.shape),
            rep((1, w3.shape[0])),
        ],
        out_specs=pl.BlockSpec((1, 1, Cout), lambda b, n: (b, 0, 0)),
        out_shape=jax.ShapeDtypeStruct((B, 1, Cout), jnp.float32),
        scratch_shapes=[pltpu.VMEM((1, Cout), jnp.bfloat16)],
        compiler_params=pltpu.CompilerParams(
            dimension_semantics=("parallel", "arbitrary")),
    )(xyz, features, w1x, w1f, b1r, w2, b2r, w3, b3r)
    return out.reshape(B, Cout)


# speedup vs baseline: 1.0366x; 1.0366x over previous
import functools

import jax
import jax.numpy as jnp
from jax.experimental import pallas as pl
from jax.experimental.pallas import tpu as pltpu


TILE_N = 1024


def _body(xyz_ref, feat_ref, w1x_ref, w1f_ref, b1_ref, w2_ref, b2_ref,
          w3_ref, b3_ref, out_ref, acc_ref):
    n = pl.program_id(1)
    num_n = pl.num_programs(1)
    x = feat_ref[0].astype(jnp.bfloat16)
    xyzt = xyz_ref[0].astype(jnp.bfloat16)

    h1 = jnp.dot(x, w1f_ref[...], preferred_element_type=jnp.float32)
    h1 += jnp.dot(xyzt, w1x_ref[...], preferred_element_type=jnp.float32)
    h1 = jnp.maximum(h1.astype(jnp.bfloat16) + b1_ref[...], 0.0)

    h2 = jnp.dot(h1, w2_ref[...], preferred_element_type=jnp.float32)
    h2 = jnp.maximum(h2.astype(jnp.bfloat16) + b2_ref[...], 0.0)

    h3 = jnp.dot(h2, w3_ref[...], preferred_element_type=jnp.float32)

    tile_max = jnp.max(h3, axis=0, keepdims=True).astype(jnp.bfloat16)

    @pl.when(n == 0)
    def _init():
        acc_ref[...] = tile_max

    @pl.when(n != 0)
    def _acc():
        acc_ref[...] = jnp.maximum(acc_ref[...], tile_max)

    @pl.when(n == num_n - 1)
    def _finish():
        m = acc_ref[...].astype(jnp.float32)
        out_ref[0] = jnp.maximum(m + b3_ref[...], 0.0)


@functools.partial(jax.jit, static_argnames=())
def kernel(xyz, features, W1, b1, W2, b2, W3, b3):
    B, N, C = features.shape
    Cout = W3.shape[0]
    T = TILE_N

    w1x = jnp.transpose(W1[:, :3]).astype(jnp.bfloat16)
    w1f = jnp.transpose(W1[:, 3:]).astype(jnp.bfloat16)
    w2 = jnp.transpose(W2).astype(jnp.bfloat16)
    w3 = jnp.transpose(W3).astype(jnp.bfloat16)
    b1r = b1.reshape(1, -1).astype(jnp.bfloat16)
    b2r = b2.reshape(1, -1).astype(jnp.bfloat16)
    b3r = b3.reshape(1, -1)

    rep = lambda shape: pl.BlockSpec(shape, lambda b, n: (0,) * len(shape))

    out = pl.pallas_call(
        _body,
        grid=(B, N // T),
        in_specs=[
            pl.BlockSpec((1, T, 3), lambda b, n: (b, n, 0)),
            pl.BlockSpec((1, T, C), lambda b, n: (b, n, 0)),
            rep(w1x.shape),
            rep(w1f.shape),
            rep((1, w1f.shape[1])),
            rep(w2.shape),
            rep((1, w2.shape[1])),
            rep(w3.shape),
            rep((1, w3.shape[1])),
        ],
        out_specs=pl.BlockSpec((1, 1, Cout), lambda b, n: (b, 0, 0)),
        out_shape=jax.ShapeDtypeStruct((B, 1, Cout), jnp.float32),
        scratch_shapes=[pltpu.VMEM((1, Cout), jnp.bfloat16)],
        compiler_params=pltpu.CompilerParams(
            dimension_semantics=("parallel", "arbitrary")),
    )(xyz, features, w1x, w1f, b1r, w2, b2r, w3, b3r)
    return out.reshape(B, Cout)


# speedup vs baseline: 1.2928x; 1.2471x over previous
import functools

import jax
import jax.numpy as jnp
from jax.experimental import pallas as pl
from jax.experimental.pallas import tpu as pltpu


TILE_N = 2048


def _body(xyz_ref, feat_ref, w1x_ref, w1f_ref, b1_ref, w2_ref, b2_ref,
          w3_ref, b3_ref, out_ref, acc_ref):
    n = pl.program_id(1)
    num_n = pl.num_programs(1)
    x = feat_ref[0].astype(jnp.bfloat16)
    xyzt = xyz_ref[0].astype(jnp.bfloat16)

    h1 = jnp.dot(x, w1f_ref[...],
                 preferred_element_type=jnp.float32).astype(jnp.bfloat16)
    h1 += xyzt[:, 0:1] * w1x_ref[0:1, :]
    h1 += xyzt[:, 1:2] * w1x_ref[1:2, :]
    h1 += xyzt[:, 2:3] * w1x_ref[2:3, :]
    h1 = jnp.maximum(h1 + b1_ref[...], 0.0)

    h2 = jnp.dot(h1, w2_ref[...], preferred_element_type=jnp.float32)
    h2 = jnp.maximum(h2.astype(jnp.bfloat16) + b2_ref[...], 0.0)

    h3 = jnp.dot(h2, w3_ref[...], preferred_element_type=jnp.float32)

    tile_max = jnp.max(h3, axis=0, keepdims=True).astype(jnp.bfloat16)

    @pl.when(n == 0)
    def _init():
        acc_ref[...] = tile_max

    @pl.when(n != 0)
    def _acc():
        acc_ref[...] = jnp.maximum(acc_ref[...], tile_max)

    @pl.when(n == num_n - 1)
    def _finish():
        m = acc_ref[...].astype(jnp.float32)
        out_ref[0] = jnp.maximum(m + b3_ref[...], 0.0)


@functools.partial(jax.jit, static_argnames=())
def kernel(xyz, features, W1, b1, W2, b2, W3, b3):
    B, N, C = features.shape
    Cout = W3.shape[0]
    T = TILE_N

    w1x = jnp.transpose(W1[:, :3]).astype(jnp.bfloat16)
    w1f = jnp.transpose(W1[:, 3:]).astype(jnp.bfloat16)
    w2 = jnp.transpose(W2).astype(jnp.bfloat16)
    w3 = jnp.transpose(W3).astype(jnp.bfloat16)
    b1r = b1.reshape(1, -1).astype(jnp.bfloat16)
    b2r = b2.reshape(1, -1).astype(jnp.bfloat16)
    b3r = b3.reshape(1, -1)

    rep = lambda shape: pl.BlockSpec(shape, lambda b, n: (0,) * len(shape))

    out = pl.pallas_call(
        _body,
        grid=(B, N // T),
        in_specs=[
            pl.BlockSpec((1, T, 3), lambda b, n: (b, n, 0)),
            pl.BlockSpec((1, T, C), lambda b, n: (b, n, 0)),
            rep(w1x.shape),
            rep(w1f.shape),
            rep((1, w1f.shape[1])),
            rep(w2.shape),
            rep((1, w2.shape[1])),
            rep(w3.shape),
            rep((1, w3.shape[1])),
        ],
        out_specs=pl.BlockSpec((1, 1, Cout), lambda b, n: (b, 0, 0)),
        out_shape=jax.ShapeDtypeStruct((B, 1, Cout), jnp.float32),
        scratch_shapes=[pltpu.VMEM((1, Cout), jnp.bfloat16)],
        compiler_params=pltpu.CompilerParams(
            dimension_semantics=("parallel", "arbitrary")),
    )(xyz, features, w1x, w1f, b1r, w2, b2r, w3, b3r)
    return out.reshape(B, Cout)


# speedup vs baseline: 1.3499x; 1.0442x over previous
import functools

import jax
import jax.numpy as jnp
from jax.experimental import pallas as pl
from jax.experimental.pallas import tpu as pltpu


TILE_N = 4096


def _body(xyz_ref, feat_ref, w1x_ref, w1f_ref, b1_ref, w2_ref, b2_ref,
          w3_ref, b3_ref, out_ref, acc_ref):
    n = pl.program_id(1)
    num_n = pl.num_programs(1)
    x = feat_ref[0].astype(jnp.bfloat16)
    xyzt = xyz_ref[0].astype(jnp.bfloat16)

    h1 = jnp.dot(x, w1f_ref[...],
                 preferred_element_type=jnp.float32).astype(jnp.bfloat16)
    h1 += xyzt[:, 0:1] * w1x_ref[0:1, :]
    h1 += xyzt[:, 1:2] * w1x_ref[1:2, :]
    h1 += xyzt[:, 2:3] * w1x_ref[2:3, :]
    h1 = jnp.maximum(h1 + b1_ref[...], 0.0)

    h2 = jnp.dot(h1, w2_ref[...], preferred_element_type=jnp.float32)
    h2 = jnp.maximum(h2.astype(jnp.bfloat16) + b2_ref[...], 0.0)

    h3 = jnp.dot(h2, w3_ref[...], preferred_element_type=jnp.float32)

    tile_max = jnp.max(h3, axis=0, keepdims=True).astype(jnp.bfloat16)

    @pl.when(n == 0)
    def _init():
        acc_ref[...] = tile_max

    @pl.when(n != 0)
    def _acc():
        acc_ref[...] = jnp.maximum(acc_ref[...], tile_max)

    @pl.when(n == num_n - 1)
    def _finish():
        m = acc_ref[...].astype(jnp.float32)
        out_ref[0] = jnp.maximum(m + b3_ref[...], 0.0)


@functools.partial(jax.jit, static_argnames=())
def kernel(xyz, features, W1, b1, W2, b2, W3, b3):
    B, N, C = features.shape
    Cout = W3.shape[0]
    T = TILE_N

    w1x = jnp.transpose(W1[:, :3]).astype(jnp.bfloat16)
    w1f = jnp.transpose(W1[:, 3:]).astype(jnp.bfloat16)
    w2 = jnp.transpose(W2).astype(jnp.bfloat16)
    w3 = jnp.transpose(W3).astype(jnp.bfloat16)
    b1r = b1.reshape(1, -1).astype(jnp.bfloat16)
    b2r = b2.reshape(1, -1).astype(jnp.bfloat16)
    b3r = b3.reshape(1, -1)

    rep = lambda shape: pl.BlockSpec(shape, lambda b, n: (0,) * len(shape))

    out = pl.pallas_call(
        _body,
        grid=(B, N // T),
        in_specs=[
            pl.BlockSpec((1, T, 3), lambda b, n: (b, n, 0)),
            pl.BlockSpec((1, T, C), lambda b, n: (b, n, 0)),
            rep(w1x.shape),
            rep(w1f.shape),
            rep((1, w1f.shape[1])),
            rep(w2.shape),
            rep((1, w2.shape[1])),
            rep(w3.shape),
            rep((1, w3.shape[1])),
        ],
        out_specs=pl.BlockSpec((1, 1, Cout), lambda b, n: (b, 0, 0)),
        out_shape=jax.ShapeDtypeStruct((B, 1, Cout), jnp.float32),
        scratch_shapes=[pltpu.VMEM((1, Cout), jnp.bfloat16)],
        compiler_params=pltpu.CompilerParams(
            dimension_semantics=("parallel", "arbitrary")),
    )(xyz, features, w1x, w1f, b1r, w2, b2r, w3, b3r)
    return out.reshape(B, Cout)


# speedup vs baseline: 1.4056x; 1.0413x over previous
import functools

import jax
import jax.numpy as jnp
from jax.experimental import pallas as pl
from jax.experimental.pallas import tpu as pltpu


TILE_N = 4096


def _body(xyz_ref, feat_ref, w1x_ref, w1f_src_ref, b1_ref, w2_src_ref,
          b2_ref, w3_src_ref, b3_ref, out_ref,
          acc_ref, w1f_ref, w2_ref, w3_ref):
    b = pl.program_id(0)
    n = pl.program_id(1)
    num_n = pl.num_programs(1)

    @pl.when((b == 0) & (n == 0))
    def _prep_weights():
        w1f_ref[...] = jnp.transpose(w1f_src_ref[...]).astype(jnp.bfloat16)
        w2_ref[...] = jnp.transpose(w2_src_ref[...]).astype(jnp.bfloat16)
        w3_ref[...] = jnp.transpose(w3_src_ref[...]).astype(jnp.bfloat16)

    x = feat_ref[0].astype(jnp.bfloat16)
    xyzt = xyz_ref[0].astype(jnp.bfloat16)
    b1 = b1_ref[...].astype(jnp.bfloat16)
    b2 = b2_ref[...].astype(jnp.bfloat16)

    h1 = jnp.dot(x, w1f_ref[...],
                 preferred_element_type=jnp.float32).astype(jnp.bfloat16)
    h1 += xyzt[:, 0:1] * w1x_ref[0:1, :]
    h1 += xyzt[:, 1:2] * w1x_ref[1:2, :]
    h1 += xyzt[:, 2:3] * w1x_ref[2:3, :]
    h1 = jnp.maximum(h1 + b1, 0.0)

    h2 = jnp.dot(h1, w2_ref[...],
                 preferred_element_type=jnp.float32).astype(jnp.bfloat16)
    h2 = jnp.maximum(h2 + b2, 0.0)

    h3 = jnp.dot(h2, w3_ref[...], preferred_element_type=jnp.float32)

    tile_max = jnp.max(h3, axis=0, keepdims=True).astype(jnp.bfloat16)

    @pl.when(n == 0)
    def _init():
        acc_ref[...] = tile_max

    @pl.when(n != 0)
    def _acc():
        acc_ref[...] = jnp.maximum(acc_ref[...], tile_max)

    @pl.when(n == num_n - 1)
    def _finish():
        m = acc_ref[...].astype(jnp.float32)
        out_ref[0] = jnp.maximum(m + b3_ref[...], 0.0)


@functools.partial(jax.jit, static_argnames=())
def kernel(xyz, features, W1, b1, W2, b2, W3, b3):
    B, N, C = features.shape
    Cout = W3.shape[0]
    T = TILE_N

    w1x = jnp.transpose(W1[:, :3]).astype(jnp.bfloat16)
    w1f_src = W1[:, 3:]

    rep = lambda shape: pl.BlockSpec(shape, lambda b, n: (0,) * len(shape))

    out = pl.pallas_call(
        _body,
        grid=(B, N // T),
        in_specs=[
            pl.BlockSpec((1, T, 3), lambda b, n: (b, n, 0)),
            pl.BlockSpec((1, T, C), lambda b, n: (b, n, 0)),
            rep(w1x.shape),
            rep(w1f_src.shape),
            rep((1, W1.shape[0])),
            rep(W2.shape),
            rep((1, W2.shape[0])),
            rep(W3.shape),
            rep((1, W3.shape[0])),
        ],
        out_specs=pl.BlockSpec((1, 1, Cout), lambda b, n: (b, 0, 0)),
        out_shape=jax.ShapeDtypeStruct((B, 1, Cout), jnp.float32),
        scratch_shapes=[
            pltpu.VMEM((1, Cout), jnp.bfloat16),
            pltpu.VMEM((w1f_src.shape[1], w1f_src.shape[0]), jnp.bfloat16),
            pltpu.VMEM((W2.shape[1], W2.shape[0]), jnp.bfloat16),
            pltpu.VMEM((W3.shape[1], W3.shape[0]), jnp.bfloat16),
        ],
        compiler_params=pltpu.CompilerParams(
            dimension_semantics=("arbitrary", "arbitrary")),
    )(xyz, features, w1x, w1f_src, b1.reshape(1, -1), W2,
      b2.reshape(1, -1), W3, b3.reshape(1, -1))
    return out.reshape(B, Cout)


# speedup vs baseline: 1.4267x; 1.0150x over previous
import functools

import jax
import jax.numpy as jnp
from jax.experimental import pallas as pl
from jax.experimental.pallas import tpu as pltpu


TILE_N = 4096


def _body(xyz_ref, feat_ref, w1_src_ref, b1_ref, w2_src_ref,
          b2_ref, w3_src_ref, b3_ref, out_ref,
          acc_ref, w1x_ref, w1f_ref, w2_ref, w3_ref):
    b = pl.program_id(0)
    n = pl.program_id(1)
    num_n = pl.num_programs(1)

    @pl.when((b == 0) & (n == 0))
    def _prep_weights():
        w1t = jnp.transpose(w1_src_ref[...]).astype(jnp.bfloat16)
        w1x_ref[...] = w1t[0:3, :]
        w1f_ref[...] = w1t[3:, :]
        w2_ref[...] = jnp.transpose(w2_src_ref[...]).astype(jnp.bfloat16)
        w3_ref[...] = jnp.transpose(w3_src_ref[...]).astype(jnp.bfloat16)

    x = feat_ref[0].astype(jnp.bfloat16)
    xyzt = xyz_ref[0].astype(jnp.bfloat16)
    b1 = b1_ref[...].astype(jnp.bfloat16)
    b2 = b2_ref[...].astype(jnp.bfloat16)

    h1 = jnp.dot(x, w1f_ref[...],
                 preferred_element_type=jnp.float32).astype(jnp.bfloat16)
    h1 += xyzt[:, 0:1] * w1x_ref[0:1, :]
    h1 += xyzt[:, 1:2] * w1x_ref[1:2, :]
    h1 += xyzt[:, 2:3] * w1x_ref[2:3, :]
    h1 = jnp.maximum(h1 + b1, 0.0)

    h2 = jnp.dot(h1, w2_ref[...],
                 preferred_element_type=jnp.float32).astype(jnp.bfloat16)
    h2 = jnp.maximum(h2 + b2, 0.0)

    h3 = jnp.dot(h2, w3_ref[...], preferred_element_type=jnp.float32)

    tile_max = jnp.max(h3, axis=0, keepdims=True).astype(jnp.bfloat16)

    @pl.when(n == 0)
    def _init():
        acc_ref[...] = tile_max

    @pl.when(n != 0)
    def _acc():
        acc_ref[...] = jnp.maximum(acc_ref[...], tile_max)

    @pl.when(n == num_n - 1)
    def _finish():
        m = acc_ref[...].astype(jnp.float32)
        out_ref[0] = jnp.maximum(m + b3_ref[...], 0.0)


@functools.partial(jax.jit, static_argnames=())
def kernel(xyz, features, W1, b1, W2, b2, W3, b3):
    B, N, C = features.shape
    Cout = W3.shape[0]
    T = TILE_N

    rep = lambda shape: pl.BlockSpec(shape, lambda b, n: (0,) * len(shape))

    out = pl.pallas_call(
        _body,
        grid=(B, N // T),
        in_specs=[
            pl.BlockSpec((1, T, 3), lambda b, n: (b, n, 0)),
            pl.BlockSpec((1, T, C), lambda b, n: (b, n, 0)),
            rep(W1.shape),
            rep((1, W1.shape[0])),
            rep(W2.shape),
            rep((1, W2.shape[0])),
            rep(W3.shape),
            rep((1, W3.shape[0])),
        ],
        out_specs=pl.BlockSpec((1, 1, Cout), lambda b, n: (b, 0, 0)),
        out_shape=jax.ShapeDtypeStruct((B, 1, Cout), jnp.float32),
        scratch_shapes=[
            pltpu.VMEM((1, Cout), jnp.bfloat16),
            pltpu.VMEM((3, W1.shape[0]), jnp.bfloat16),
            pltpu.VMEM((W1.shape[1] - 3, W1.shape[0]), jnp.bfloat16),
            pltpu.VMEM((W2.shape[1], W2.shape[0]), jnp.bfloat16),
            pltpu.VMEM((W3.shape[1], W3.shape[0]), jnp.bfloat16),
        ],
        compiler_params=pltpu.CompilerParams(
            dimension_semantics=("arbitrary", "arbitrary")),
    )(xyz, features, W1, b1.reshape(1, -1), W2,
      b2.reshape(1, -1), W3, b3.reshape(1, -1))
    return out.reshape(B, Cout)
